# Initial kernel scaffold; baseline (speedup 1.0000x reference)
import jax
import jax.numpy as jnp
from jax.experimental import pallas as pl


def kernel(x, edge_index, W_in, b_in, W1, b1, W2, b2):
    raise NotImplementedError("write your pallas kernel here")


# speedup vs baseline: 9.5967x; 9.5967x over previous
import functools

import jax
import jax.numpy as jnp
from jax import lax
from jax.experimental import pallas as pl
from jax.experimental.pallas import tpu as pltpu
from jax.experimental.pallas import tpu_sc as plsc

N = 10000
NPAD = 10240
D = 256
HALF = 128
DOUT = 16
E = 160000
EP = 163840
STRIPE = NPAD // 16
CH = 128

def _mesh():
    return plsc.VectorSubcoreMesh(core_axis_name="c", subcore_axis_name="s")


def _deg_body(dst_hbm, zeros_hbm, ones_hbm, out_hbm, idx_v, ones_v, deg_sh, sem):
    c = lax.axis_index("c")
    s = lax.axis_index("s")
    pltpu.sync_copy(zeros_hbm.at[pl.ds(s * STRIPE, STRIPE)],
                    deg_sh.at[pl.ds(s * STRIPE, STRIPE)])
    pltpu.sync_copy(ones_hbm, ones_v)
    pltpu.sync_copy(dst_hbm.at[c, s], idx_v)
    plsc.subcore_barrier()

    def body(j, carry):
        pltpu.sync_copy(ones_v, deg_sh.at[idx_v.at[j]], add=True)
        return carry

    lax.fori_loop(0, 40, body, 0)
    plsc.subcore_barrier()
    pltpu.sync_copy(deg_sh.at[pl.ds(s * STRIPE, STRIPE)],
                    out_hbm.at[c, pl.ds(s * STRIPE, STRIPE)])


@functools.lru_cache(maxsize=None)
def _deg_call():
    return pl.kernel(
        _deg_body,
        mesh=_mesh(),
        out_type=jax.ShapeDtypeStruct((2, NPAD), jnp.float32),
        scratch_types=[
            pltpu.VMEM((40, CH), jnp.int32),
            pltpu.VMEM((CH,), jnp.float32),
            pltpu.VMEM_SHARED((NPAD,), jnp.float32),
            pltpu.SemaphoreType.DMA,
        ],
    )


def _conv1_body(g1_hbm, src_hbm, dst_hbm, out_hbm, srcv, dstv, rows, acc_sh, sem):
    c = lax.axis_index("c")
    s = lax.axis_index("s")
    base = c * NPAD + s * STRIPE
    pltpu.sync_copy(g1_hbm.at[pl.ds(base, STRIPE)],
                    acc_sh.at[pl.ds(s * STRIPE, STRIPE)])
    pltpu.sync_copy(src_hbm.at[c, s], srcv)
    pltpu.sync_copy(dst_hbm.at[s], dstv)
    plsc.subcore_barrier()

    def body(j, carry):
        pltpu.async_copy(g1_hbm.at[srcv.at[j]], rows, sem).wait()
        pltpu.sync_copy(rows, acc_sh.at[dstv.at[j]], add=True)
        return carry

    lax.fori_loop(0, EP // 16 // CH, body, 0)
    plsc.subcore_barrier()
    pltpu.sync_copy(acc_sh.at[pl.ds(s * STRIPE, STRIPE)],
                    out_hbm.at[pl.ds(base, STRIPE)])


@functools.lru_cache(maxsize=None)
def _conv1_call():
    return pl.kernel(
        _conv1_body,
        mesh=_mesh(),
        out_type=jax.ShapeDtypeStruct((2 * NPAD, HALF), jnp.float32),
        scratch_types=[
            pltpu.VMEM((EP // 16 // CH, CH), jnp.int32),
            pltpu.VMEM((EP // 16 // CH, CH), jnp.int32),
            pltpu.VMEM((CH, HALF), jnp.float32),
            pltpu.VMEM_SHARED((NPAD, HALF), jnp.float32),
            pltpu.SemaphoreType.DMA,
        ],
    )


def _conv2_body(g2_hbm, z16_hbm, src_hbm, dst_hbm, out_hbm, srcv, dstv, rows,
                acc_sh, sem):
    c = lax.axis_index("c")
    s = lax.axis_index("s")

    @pl.when(c == 0)
    def _():
        pltpu.sync_copy(g2_hbm.at[pl.ds(s * STRIPE, STRIPE)],
                        acc_sh.at[pl.ds(s * STRIPE, STRIPE)])

    @pl.when(c == 1)
    def _():
        pltpu.sync_copy(z16_hbm.at[pl.ds(s * STRIPE, STRIPE)],
                        acc_sh.at[pl.ds(s * STRIPE, STRIPE)])

    pltpu.sync_copy(src_hbm.at[c, s], srcv)
    pltpu.sync_copy(dst_hbm.at[c, s], dstv)
    plsc.subcore_barrier()

    def body(j, carry):
        pltpu.async_copy(g2_hbm.at[srcv.at[j]], rows, sem).wait()
        pltpu.sync_copy(rows, acc_sh.at[dstv.at[j]], add=True)
        return carry

    lax.fori_loop(0, EP // 32 // CH, body, 0)
    plsc.subcore_barrier()
    pltpu.sync_copy(acc_sh.at[pl.ds(s * STRIPE, STRIPE)],
                    out_hbm.at[c, pl.ds(s * STRIPE, STRIPE)])


@functools.lru_cache(maxsize=None)
def _conv2_call():
    return pl.kernel(
        _conv2_body,
        mesh=_mesh(),
        out_type=jax.ShapeDtypeStruct((2, NPAD, DOUT), jnp.float32),
        scratch_types=[
            pltpu.VMEM((EP // 32 // CH, CH), jnp.int32),
            pltpu.VMEM((EP // 32 // CH, CH), jnp.int32),
            pltpu.VMEM((CH, DOUT), jnp.float32),
            pltpu.VMEM_SHARED((NPAD, DOUT), jnp.float32),
            pltpu.SemaphoreType.DMA,
        ],
        compiler_params=pltpu.CompilerParams(use_tc_tiling_on_sc=False),
    )


RB = 512


def _dinv(degp_ref):
    deg = degp_ref[0] + degp_ref[1] + 1.0
    return lax.rsqrt(deg)[:, None]


def _stageB_body(x_ref, degp_ref, Win_ref, bin_ref, W1_ref, out_ref):
    dinv = _dinv(degp_ref)
    h0 = jnp.maximum(x_ref[...] @ Win_ref[...] + bin_ref[...], 0.0)
    t = (h0 @ W1_ref[...]) * dinv
    out_ref[0] = t[:, :HALF]
    out_ref[1] = t[:, HALF:]


def _stageB(x_pad, degp, W_in, b_in, W1):
    return pl.pallas_call(
        _stageB_body,
        grid=(NPAD // RB,),
        in_specs=[
            pl.BlockSpec((RB, D), lambda i: (i, 0)),
            pl.BlockSpec((2, RB), lambda i: (0, i)),
            pl.BlockSpec((D, D), lambda i: (0, 0)),
            pl.BlockSpec((1, D), lambda i: (0, 0)),
            pl.BlockSpec((D, D), lambda i: (0, 0)),
        ],
        out_specs=pl.BlockSpec((2, RB, HALF), lambda i: (0, i, 0)),
        out_shape=jax.ShapeDtypeStruct((2, NPAD, HALF), jnp.float32),
    )(x_pad, degp, W_in, b_in.reshape(1, D), W1)


def _stageD_body(agg_ref, degp_ref, b1_ref, W2_ref, out_ref):
    dinv = _dinv(degp_ref)
    agg = jnp.concatenate([agg_ref[0], agg_ref[1]], axis=1)
    h1 = jnp.maximum(agg * dinv + b1_ref[...], 0.0)
    out_ref[...] = (h1 @ W2_ref[...]) * dinv


def _stageD(agg1, degp, b1, W2):
    return pl.pallas_call(
        _stageD_body,
        grid=(NPAD // RB,),
        in_specs=[
            pl.BlockSpec((2, RB, HALF), lambda i: (0, i, 0)),
            pl.BlockSpec((2, RB), lambda i: (0, i)),
            pl.BlockSpec((1, D), lambda i: (0, 0)),
            pl.BlockSpec((D, DOUT), lambda i: (0, 0)),
        ],
        out_specs=pl.BlockSpec((RB, DOUT), lambda i: (i, 0)),
        out_shape=jax.ShapeDtypeStruct((NPAD, DOUT), jnp.float32),
    )(agg1, degp, b1.reshape(1, D), W2)


def _stageF_body(acc_ref, degp_ref, b2_ref, out_ref):
    dinv = _dinv(degp_ref)
    out_ref[...] = (acc_ref[0] + acc_ref[1]) * dinv + b2_ref[...]


def _stageF(acc, degp, b2):
    return pl.pallas_call(
        _stageF_body,
        grid=(NPAD // RB,),
        in_specs=[
            pl.BlockSpec((2, RB, DOUT), lambda i: (0, i, 0)),
            pl.BlockSpec((2, RB), lambda i: (0, i)),
            pl.BlockSpec((1, DOUT), lambda i: (0, 0)),
        ],
        out_specs=pl.BlockSpec((RB, DOUT), lambda i: (i, 0)),
        out_shape=jax.ShapeDtypeStruct((NPAD, DOUT), jnp.float32),
    )(acc, degp, b2.reshape(1, DOUT))


def kernel(x, edge_index, W_in, b_in, W1, b1, W2, b2):
    ei = edge_index.astype(jnp.int32)
    src, dst = ei[0], ei[1]
    srcp = jnp.concatenate([src, jnp.full((EP - E,), N, jnp.int32)])
    dstp = jnp.concatenate([dst, jnp.full((EP - E,), N + 200, jnp.int32)])

    x_pad = jnp.pad(x, ((0, NPAD - N), (0, 0)))

    dstA = dstp.reshape(2, 16, 40, CH)
    srcC = jnp.stack([srcp, srcp + NPAD]).reshape(2, 16, EP // 16 // CH, CH)
    dstC = dstp.reshape(16, EP // 16 // CH, CH)
    srcE = srcp.reshape(2, 16, 40, CH)

    zeros1 = jnp.zeros((NPAD,), jnp.float32)
    ones128 = jnp.ones((CH,), jnp.float32)
    zeros16 = jnp.zeros((NPAD, DOUT), jnp.float32)

    degp = _deg_call()(dstA, zeros1, ones128)
    g1 = _stageB(x_pad, degp, W_in, b_in, W1)
    agg1 = _conv1_call()(g1.reshape(2 * NPAD, HALF), srcC, dstC)
    g2 = _stageD(agg1.reshape(2, NPAD, HALF), degp, b1, W2)
    acc = _conv2_call()(g2, zeros16, srcE, dstA)
    out = _stageF(acc, degp, b2)
    return out[:N]


# speedup vs baseline: 10.9228x; 1.1382x over previous
import functools

import jax
import jax.numpy as jnp
from jax import lax
from jax.experimental import pallas as pl
from jax.experimental.pallas import tpu as pltpu
from jax.experimental.pallas import tpu_sc as plsc

N = 10000
NPAD = 10240
D = 256
HALF = 128
DOUT = 16
E = 160000
EP = 163840
STRIPE = NPAD // 16
CH = 128
CH1 = 128

def _mesh():
    return plsc.VectorSubcoreMesh(core_axis_name="c", subcore_axis_name="s")


def _deg_body(dst_hbm, zeros_hbm, ones_hbm, out_hbm, idx_v, ones_v, deg_sh, sem):
    c = lax.axis_index("c")
    s = lax.axis_index("s")
    pltpu.sync_copy(zeros_hbm.at[pl.ds(s * STRIPE, STRIPE)],
                    deg_sh.at[pl.ds(s * STRIPE, STRIPE)])
    pltpu.sync_copy(ones_hbm, ones_v)
    pltpu.sync_copy(dst_hbm.at[c, s], idx_v)
    plsc.subcore_barrier()

    def body(j, carry):
        pltpu.sync_copy(ones_v, deg_sh.at[idx_v.at[j]], add=True)
        return carry

    lax.fori_loop(0, 40, body, 0)
    plsc.subcore_barrier()
    pltpu.sync_copy(deg_sh.at[pl.ds(s * STRIPE, STRIPE)],
                    out_hbm.at[c, pl.ds(s * STRIPE, STRIPE)])


@functools.lru_cache(maxsize=None)
def _deg_call():
    return pl.kernel(
        _deg_body,
        mesh=_mesh(),
        out_type=jax.ShapeDtypeStruct((2, NPAD), jnp.float32),
        scratch_types=[
            pltpu.VMEM((40, CH), jnp.int32),
            pltpu.VMEM((CH,), jnp.float32),
            pltpu.VMEM_SHARED((NPAD,), jnp.float32),
            pltpu.SemaphoreType.DMA,
        ],
    )


def _pipelined_segsum(table_hbm, srcv, dstv, rows, acc_sh, gsems, ssems,
                      nchunks):
    del ssems
    nb = 2

    def gather(j, b):
        return pltpu.async_copy(table_hbm.at[srcv.at[j]], rows.at[b],
                                gsems[b])

    gather(0, 0)

    def body(jj, carry):
        for b in range(nb):
            j = jj * nb + b

            @pl.when(j + 1 < nchunks)
            def _():
                gather(j + 1, 1 - b)

            pltpu.make_async_copy(table_hbm.at[srcv.at[j]], rows.at[b],
                                  gsems[b]).wait()
            pltpu.sync_copy(rows.at[b], acc_sh.at[dstv.at[j]], add=True)
        return carry

    lax.fori_loop(0, nchunks // nb, body, 0)


def _conv1_body(g1_hbm, src_hbm, dst_hbm, out_hbm, ibuf, rows, acc_sh,
                gsem0, gsem1, isem0, isem1):
    c = lax.axis_index("c")
    s = lax.axis_index("s")
    base = c * NPAD + s * STRIPE
    pltpu.sync_copy(g1_hbm.at[pl.ds(base, STRIPE)],
                    acc_sh.at[pl.ds(s * STRIPE, STRIPE)])
    plsc.subcore_barrier()

    gsems = (gsem0, gsem1)
    isems = (isem0, isem1)
    nchunks = EP // 16 // CH1

    def load_idx(j, b):
        pltpu.async_copy(src_hbm.at[c, s, j], ibuf.at[b, 0], isems[b])
        pltpu.async_copy(dst_hbm.at[s, j], ibuf.at[b, 1], isems[b])

    def wait_idx(b):
        pltpu.make_async_copy(src_hbm.at[c, s, 0], ibuf.at[b, 0],
                              isems[b]).wait()
        pltpu.make_async_copy(dst_hbm.at[s, 0], ibuf.at[b, 1],
                              isems[b]).wait()

    def gather(b):
        pltpu.async_copy(g1_hbm.at[ibuf.at[b, 0]], rows.at[b], gsems[b])

    def wait_gather(b):
        pltpu.make_async_copy(g1_hbm.at[ibuf.at[b, 0]], rows.at[b],
                              gsems[b]).wait()

    load_idx(0, 0)
    load_idx(1, 1)
    wait_idx(0)
    gather(0)

    def body(jj, carry):
        for b in range(2):
            j = jj * 2 + b
            wait_gather(b)

            @pl.when(j + 1 < nchunks)
            def _():
                wait_idx(1 - b)
                gather(1 - b)

            pltpu.sync_copy(rows.at[b], acc_sh.at[ibuf.at[b, 1]], add=True)

            @pl.when(j + 2 < nchunks)
            def _():
                load_idx(j + 2, b)
        return carry

    lax.fori_loop(0, nchunks // 2, body, 0)
    plsc.subcore_barrier()
    pltpu.sync_copy(acc_sh.at[pl.ds(s * STRIPE, STRIPE)],
                    out_hbm.at[pl.ds(base, STRIPE)])


@functools.lru_cache(maxsize=None)
def _conv1_call():
    return pl.kernel(
        _conv1_body,
        mesh=_mesh(),
        out_type=jax.ShapeDtypeStruct((2 * NPAD, HALF), jnp.float32),
        scratch_types=[
            pltpu.VMEM((2, 2, CH1), jnp.int32),
            pltpu.VMEM((2, CH1, HALF), jnp.float32),
            pltpu.VMEM_SHARED((NPAD, HALF), jnp.float32),
            pltpu.SemaphoreType.DMA,
            pltpu.SemaphoreType.DMA,
            pltpu.SemaphoreType.DMA,
            pltpu.SemaphoreType.DMA,
        ],
    )


def _conv2_body(g2_hbm, z16_hbm, src_hbm, dst_hbm, out_hbm, srcv, dstv, rows,
                acc_sh, gsem0, gsem1, ssem0, ssem1):
    c = lax.axis_index("c")
    s = lax.axis_index("s")

    @pl.when(c == 0)
    def _():
        pltpu.sync_copy(g2_hbm.at[pl.ds(s * STRIPE, STRIPE)],
                        acc_sh.at[pl.ds(s * STRIPE, STRIPE)])

    @pl.when(c == 1)
    def _():
        pltpu.sync_copy(z16_hbm.at[pl.ds(s * STRIPE, STRIPE)],
                        acc_sh.at[pl.ds(s * STRIPE, STRIPE)])

    pltpu.sync_copy(src_hbm.at[c, s], srcv)
    pltpu.sync_copy(dst_hbm.at[c, s], dstv)
    plsc.subcore_barrier()
    _pipelined_segsum(g2_hbm, srcv, dstv, rows, acc_sh, (gsem0, gsem1),
                      (ssem0, ssem1), EP // 32 // CH)
    plsc.subcore_barrier()
    pltpu.sync_copy(acc_sh.at[pl.ds(s * STRIPE, STRIPE)],
                    out_hbm.at[c, pl.ds(s * STRIPE, STRIPE)])


@functools.lru_cache(maxsize=None)
def _conv2_call():
    return pl.kernel(
        _conv2_body,
        mesh=_mesh(),
        out_type=jax.ShapeDtypeStruct((2, NPAD, DOUT), jnp.float32),
        scratch_types=[
            pltpu.VMEM((EP // 32 // CH, CH), jnp.int32),
            pltpu.VMEM((EP // 32 // CH, CH), jnp.int32),
            pltpu.VMEM((2, CH, DOUT), jnp.float32),
            pltpu.VMEM_SHARED((NPAD, DOUT), jnp.float32),
            pltpu.SemaphoreType.DMA,
            pltpu.SemaphoreType.DMA,
            pltpu.SemaphoreType.DMA,
            pltpu.SemaphoreType.DMA,
        ],
        compiler_params=pltpu.CompilerParams(use_tc_tiling_on_sc=False),
    )


RB = 512


def _dinv(degp_ref):
    deg = degp_ref[0] + degp_ref[1] + 1.0
    return lax.rsqrt(deg)[:, None]


def _stageB_body(x_ref, degp_ref, Win_ref, bin_ref, W1_ref, out_ref):
    dinv = _dinv(degp_ref)
    h0 = jnp.maximum(x_ref[...] @ Win_ref[...] + bin_ref[...], 0.0)
    t = (h0 @ W1_ref[...]) * dinv
    out_ref[0] = t[:, :HALF]
    out_ref[1] = t[:, HALF:]


def _stageB(x_pad, degp, W_in, b_in, W1):
    return pl.pallas_call(
        _stageB_body,
        grid=(NPAD // RB,),
        in_specs=[
            pl.BlockSpec((RB, D), lambda i: (i, 0)),
            pl.BlockSpec((2, RB), lambda i: (0, i)),
            pl.BlockSpec((D, D), lambda i: (0, 0)),
            pl.BlockSpec((1, D), lambda i: (0, 0)),
            pl.BlockSpec((D, D), lambda i: (0, 0)),
        ],
        out_specs=pl.BlockSpec((2, RB, HALF), lambda i: (0, i, 0)),
        out_shape=jax.ShapeDtypeStruct((2, NPAD, HALF), jnp.float32),
    )(x_pad, degp, W_in, b_in.reshape(1, D), W1)


def _stageD_body(agg_ref, degp_ref, b1_ref, W2_ref, out_ref):
    dinv = _dinv(degp_ref)
    agg = jnp.concatenate([agg_ref[0], agg_ref[1]], axis=1)
    h1 = jnp.maximum(agg * dinv + b1_ref[...], 0.0)
    out_ref[...] = (h1 @ W2_ref[...]) * dinv


def _stageD(agg1, degp, b1, W2):
    return pl.pallas_call(
        _stageD_body,
        grid=(NPAD // RB,),
        in_specs=[
            pl.BlockSpec((2, RB, HALF), lambda i: (0, i, 0)),
            pl.BlockSpec((2, RB), lambda i: (0, i)),
            pl.BlockSpec((1, D), lambda i: (0, 0)),
            pl.BlockSpec((D, DOUT), lambda i: (0, 0)),
        ],
        out_specs=pl.BlockSpec((RB, DOUT), lambda i: (i, 0)),
        out_shape=jax.ShapeDtypeStruct((NPAD, DOUT), jnp.float32),
    )(agg1, degp, b1.reshape(1, D), W2)


def _stageF_body(acc_ref, degp_ref, b2_ref, out_ref):
    dinv = _dinv(degp_ref)
    out_ref[...] = (acc_ref[0] + acc_ref[1]) * dinv + b2_ref[...]


def _stageF(acc, degp, b2):
    return pl.pallas_call(
        _stageF_body,
        grid=(NPAD // RB,),
        in_specs=[
            pl.BlockSpec((2, RB, DOUT), lambda i: (0, i, 0)),
            pl.BlockSpec((2, RB), lambda i: (0, i)),
            pl.BlockSpec((1, DOUT), lambda i: (0, 0)),
        ],
        out_specs=pl.BlockSpec((RB, DOUT), lambda i: (i, 0)),
        out_shape=jax.ShapeDtypeStruct((NPAD, DOUT), jnp.float32),
    )(acc, degp, b2.reshape(1, DOUT))


def kernel(x, edge_index, W_in, b_in, W1, b1, W2, b2):
    ei = edge_index.astype(jnp.int32)
    src, dst = ei[0], ei[1]
    srcp = jnp.concatenate([src, jnp.full((EP - E,), N, jnp.int32)])
    dstp = jnp.concatenate([dst, jnp.full((EP - E,), N + 200, jnp.int32)])

    x_pad = jnp.pad(x, ((0, NPAD - N), (0, 0)))

    dstA = dstp.reshape(2, 16, 40, CH)
    srcC = jnp.stack([srcp, srcp + NPAD]).reshape(2, 16, EP // 16 // CH1, CH1)
    dstC = dstp.reshape(16, EP // 16 // CH1, CH1)
    srcE = srcp.reshape(2, 16, 40, CH)

    zeros1 = jnp.zeros((NPAD,), jnp.float32)
    ones128 = jnp.ones((CH,), jnp.float32)
    zeros16 = jnp.zeros((NPAD, DOUT), jnp.float32)

    degp = _deg_call()(dstA, zeros1, ones128)
    g1 = _stageB(x_pad, degp, W_in, b_in, W1)
    agg1 = _conv1_call()(g1.reshape(2 * NPAD, HALF), srcC, dstC)
    g2 = _stageD(agg1.reshape(2, NPAD, HALF), degp, b1, W2)
    acc = _conv2_call()(g2, zeros16, srcE, dstA)
    out = _stageF(acc, degp, b2)
    return out[:N]


# speedup vs baseline: 10.9677x; 1.0041x over previous
import functools

import jax
import jax.numpy as jnp
from jax import lax
from jax.experimental import pallas as pl
from jax.experimental.pallas import tpu as pltpu
from jax.experimental.pallas import tpu_sc as plsc

N = 10000
NPAD = 10240
D = 256
HALF = 128
DOUT = 16
E = 160000
EP = 163840
STRIPE = NPAD // 16
CH = 128
CH1 = 32

def _mesh():
    return plsc.VectorSubcoreMesh(core_axis_name="c", subcore_axis_name="s")


def _deg_body(dst_hbm, zeros_hbm, ones_hbm, out_hbm, idx_v, ones_v, deg_sh, sem):
    c = lax.axis_index("c")
    s = lax.axis_index("s")
    pltpu.sync_copy(zeros_hbm.at[pl.ds(s * STRIPE, STRIPE)],
                    deg_sh.at[pl.ds(s * STRIPE, STRIPE)])
    pltpu.sync_copy(ones_hbm, ones_v)
    pltpu.sync_copy(dst_hbm.at[c, s], idx_v)
    plsc.subcore_barrier()

    def body(j, carry):
        pltpu.sync_copy(ones_v, deg_sh.at[idx_v.at[j]], add=True)
        return carry

    lax.fori_loop(0, 40, body, 0)
    plsc.subcore_barrier()
    pltpu.sync_copy(deg_sh.at[pl.ds(s * STRIPE, STRIPE)],
                    out_hbm.at[c, pl.ds(s * STRIPE, STRIPE)])


@functools.lru_cache(maxsize=None)
def _deg_call():
    return pl.kernel(
        _deg_body,
        mesh=_mesh(),
        out_type=jax.ShapeDtypeStruct((2, NPAD), jnp.float32),
        scratch_types=[
            pltpu.VMEM((40, CH), jnp.int32),
            pltpu.VMEM((CH,), jnp.float32),
            pltpu.VMEM_SHARED((NPAD,), jnp.float32),
            pltpu.SemaphoreType.DMA,
        ],
    )


def _pipelined_segsum(table_hbm, srcv, dstv, rows, acc_sh, gsems, ssems,
                      nchunks):
    del ssems
    nb = 2

    def gather(j, b):
        return pltpu.async_copy(table_hbm.at[srcv.at[j]], rows.at[b],
                                gsems[b])

    gather(0, 0)

    def body(jj, carry):
        for b in range(nb):
            j = jj * nb + b

            @pl.when(j + 1 < nchunks)
            def _():
                gather(j + 1, 1 - b)

            pltpu.make_async_copy(table_hbm.at[srcv.at[j]], rows.at[b],
                                  gsems[b]).wait()
            pltpu.sync_copy(rows.at[b], acc_sh.at[dstv.at[j]], add=True)
        return carry

    lax.fori_loop(0, nchunks // nb, body, 0)


NROW = 8
NIDX = 16


def _conv1_body(g1_hbm, src_hbm, dst_hbm, out_hbm, ibuf, rows, acc_sh, *sems):
    gsems, isems = sems[:NROW], sems[NROW:]
    c = lax.axis_index("c")
    s = lax.axis_index("s")
    base = c * NPAD + s * STRIPE
    pltpu.sync_copy(g1_hbm.at[pl.ds(base, STRIPE)],
                    acc_sh.at[pl.ds(s * STRIPE, STRIPE)])
    plsc.subcore_barrier()

    nchunks = EP // 16 // CH1

    def load_idx(j, b):
        pltpu.async_copy(src_hbm.at[c, s, j], ibuf.at[b, 0], isems[b])
        pltpu.async_copy(dst_hbm.at[s, j], ibuf.at[b, 1], isems[b])

    def wait_idx(b):
        pltpu.make_async_copy(src_hbm.at[c, s, 0], ibuf.at[b, 0],
                              isems[b]).wait()
        pltpu.make_async_copy(dst_hbm.at[s, 0], ibuf.at[b, 1],
                              isems[b]).wait()

    def gather(br, bi):
        pltpu.async_copy(g1_hbm.at[ibuf.at[bi, 0]], rows.at[br], gsems[br])

    def wait_gather(br, bi):
        pltpu.make_async_copy(g1_hbm.at[ibuf.at[bi, 0]], rows.at[br],
                              gsems[br]).wait()

    for t in range(NIDX):
        load_idx(t, t)
    for t in range(NROW):
        wait_idx(t)
        gather(t, t)

    def body(jj, carry):
        for u in range(NIDX):
            j = jj * NIDX + u
            br, bi, bn = u % NROW, u, (u + NROW) % NIDX
            wait_gather(br, bi)
            pltpu.sync_copy(rows.at[br], acc_sh.at[ibuf.at[bi, 1]], add=True)

            @pl.when(j + NROW < nchunks)
            def _():
                wait_idx(bn)
                gather(br, bn)

            @pl.when(j + NIDX < nchunks)
            def _():
                load_idx(j + NIDX, bi)
        return carry

    lax.fori_loop(0, nchunks // NIDX, body, 0)
    plsc.subcore_barrier()
    pltpu.sync_copy(acc_sh.at[pl.ds(s * STRIPE, STRIPE)],
                    out_hbm.at[pl.ds(base, STRIPE)])


@functools.lru_cache(maxsize=None)
def _conv1_call():
    return pl.kernel(
        _conv1_body,
        mesh=_mesh(),
        out_type=jax.ShapeDtypeStruct((2 * NPAD, HALF), jnp.float32),
        scratch_types=[
            pltpu.VMEM((NIDX, 2, CH1), jnp.int32),
            pltpu.VMEM((NROW, CH1, HALF), jnp.float32),
            pltpu.VMEM_SHARED((NPAD, HALF), jnp.float32),
        ] + [pltpu.SemaphoreType.DMA] * (NROW + NIDX),
    )


def _conv2_body(g2_hbm, z16_hbm, src_hbm, dst_hbm, out_hbm, srcv, dstv, rows,
                acc_sh, gsem0, gsem1, ssem0, ssem1):
    c = lax.axis_index("c")
    s = lax.axis_index("s")

    @pl.when(c == 0)
    def _():
        pltpu.sync_copy(g2_hbm.at[pl.ds(s * STRIPE, STRIPE)],
                        acc_sh.at[pl.ds(s * STRIPE, STRIPE)])

    @pl.when(c == 1)
    def _():
        pltpu.sync_copy(z16_hbm.at[pl.ds(s * STRIPE, STRIPE)],
                        acc_sh.at[pl.ds(s * STRIPE, STRIPE)])

    pltpu.sync_copy(src_hbm.at[c, s], srcv)
    pltpu.sync_copy(dst_hbm.at[c, s], dstv)
    plsc.subcore_barrier()
    _pipelined_segsum(g2_hbm, srcv, dstv, rows, acc_sh, (gsem0, gsem1),
                      (ssem0, ssem1), EP // 32 // CH)
    plsc.subcore_barrier()
    pltpu.sync_copy(acc_sh.at[pl.ds(s * STRIPE, STRIPE)],
                    out_hbm.at[c, pl.ds(s * STRIPE, STRIPE)])


@functools.lru_cache(maxsize=None)
def _conv2_call():
    return pl.kernel(
        _conv2_body,
        mesh=_mesh(),
        out_type=jax.ShapeDtypeStruct((2, NPAD, DOUT), jnp.float32),
        scratch_types=[
            pltpu.VMEM((EP // 32 // CH, CH), jnp.int32),
            pltpu.VMEM((EP // 32 // CH, CH), jnp.int32),
            pltpu.VMEM((2, CH, DOUT), jnp.float32),
            pltpu.VMEM_SHARED((NPAD, DOUT), jnp.float32),
            pltpu.SemaphoreType.DMA,
            pltpu.SemaphoreType.DMA,
            pltpu.SemaphoreType.DMA,
            pltpu.SemaphoreType.DMA,
        ],
        compiler_params=pltpu.CompilerParams(use_tc_tiling_on_sc=False),
    )


RB = 512


def _dinv(degp_ref):
    deg = degp_ref[0] + degp_ref[1] + 1.0
    return lax.rsqrt(deg)[:, None]


def _stageB_body(x_ref, degp_ref, Win_ref, bin_ref, W1_ref, out_ref):
    dinv = _dinv(degp_ref)
    h0 = jnp.maximum(x_ref[...] @ Win_ref[...] + bin_ref[...], 0.0)
    t = (h0 @ W1_ref[...]) * dinv
    out_ref[0] = t[:, :HALF]
    out_ref[1] = t[:, HALF:]


def _stageB(x_pad, degp, W_in, b_in, W1):
    return pl.pallas_call(
        _stageB_body,
        grid=(NPAD // RB,),
        in_specs=[
            pl.BlockSpec((RB, D), lambda i: (i, 0)),
            pl.BlockSpec((2, RB), lambda i: (0, i)),
            pl.BlockSpec((D, D), lambda i: (0, 0)),
            pl.BlockSpec((1, D), lambda i: (0, 0)),
            pl.BlockSpec((D, D), lambda i: (0, 0)),
        ],
        out_specs=pl.BlockSpec((2, RB, HALF), lambda i: (0, i, 0)),
        out_shape=jax.ShapeDtypeStruct((2, NPAD, HALF), jnp.float32),
    )(x_pad, degp, W_in, b_in.reshape(1, D), W1)


def _stageD_body(agg_ref, degp_ref, b1_ref, W2_ref, out_ref):
    dinv = _dinv(degp_ref)
    agg = jnp.concatenate([agg_ref[0], agg_ref[1]], axis=1)
    h1 = jnp.maximum(agg * dinv + b1_ref[...], 0.0)
    out_ref[...] = (h1 @ W2_ref[...]) * dinv


def _stageD(agg1, degp, b1, W2):
    return pl.pallas_call(
        _stageD_body,
        grid=(NPAD // RB,),
        in_specs=[
            pl.BlockSpec((2, RB, HALF), lambda i: (0, i, 0)),
            pl.BlockSpec((2, RB), lambda i: (0, i)),
            pl.BlockSpec((1, D), lambda i: (0, 0)),
            pl.BlockSpec((D, DOUT), lambda i: (0, 0)),
        ],
        out_specs=pl.BlockSpec((RB, DOUT), lambda i: (i, 0)),
        out_shape=jax.ShapeDtypeStruct((NPAD, DOUT), jnp.float32),
    )(agg1, degp, b1.reshape(1, D), W2)


def _stageF_body(acc_ref, degp_ref, b2_ref, out_ref):
    dinv = _dinv(degp_ref)
    out_ref[...] = (acc_ref[0] + acc_ref[1]) * dinv + b2_ref[...]


def _stageF(acc, degp, b2):
    return pl.pallas_call(
        _stageF_body,
        grid=(NPAD // RB,),
        in_specs=[
            pl.BlockSpec((2, RB, DOUT), lambda i: (0, i, 0)),
            pl.BlockSpec((2, RB), lambda i: (0, i)),
            pl.BlockSpec((1, DOUT), lambda i: (0, 0)),
        ],
        out_specs=pl.BlockSpec((RB, DOUT), lambda i: (i, 0)),
        out_shape=jax.ShapeDtypeStruct((NPAD, DOUT), jnp.float32),
    )(acc, degp, b2.reshape(1, DOUT))


def kernel(x, edge_index, W_in, b_in, W1, b1, W2, b2):
    ei = edge_index.astype(jnp.int32)
    src, dst = ei[0], ei[1]
    srcp = jnp.concatenate([src, jnp.full((EP - E,), N, jnp.int32)])
    dstp = jnp.concatenate([dst, jnp.full((EP - E,), N + 200, jnp.int32)])

    x_pad = jnp.pad(x, ((0, NPAD - N), (0, 0)))

    dstA = dstp.reshape(2, 16, 40, CH)
    srcC = jnp.stack([srcp, srcp + NPAD]).reshape(2, 16, EP // 16 // CH1, CH1)
    dstC = dstp.reshape(16, EP // 16 // CH1, CH1)
    srcE = srcp.reshape(2, 16, 40, CH)

    zeros1 = jnp.zeros((NPAD,), jnp.float32)
    ones128 = jnp.ones((CH,), jnp.float32)
    zeros16 = jnp.zeros((NPAD, DOUT), jnp.float32)

    degp = _deg_call()(dstA, zeros1, ones128)
    g1 = _stageB(x_pad, degp, W_in, b_in, W1)
    agg1 = _conv1_call()(g1.reshape(2 * NPAD, HALF), srcC, dstC)
    g2 = _stageD(agg1.reshape(2, NPAD, HALF), degp, b1, W2)
    acc = _conv2_call()(g2, zeros16, srcE, dstA)
    out = _stageF(acc, degp, b2)
    return out[:N]


# speedup vs baseline: 11.4807x; 1.0468x over previous
import functools

import jax
import jax.numpy as jnp
from jax import lax
from jax.experimental import pallas as pl
from jax.experimental.pallas import tpu as pltpu
from jax.experimental.pallas import tpu_sc as plsc

N = 10000
NPAD = 10240
D = 256
HALF = 128
DOUT = 16
E = 160000
EP = 163840
STRIPE = NPAD // 16
CH = 128
CH1 = 128

def _mesh():
    return plsc.VectorSubcoreMesh(core_axis_name="c", subcore_axis_name="s")


def _deg_body(dst_hbm, zeros_hbm, ones_hbm, out_hbm, idx_v, ones_v, deg_sh, sem):
    c = lax.axis_index("c")
    s = lax.axis_index("s")
    pltpu.sync_copy(zeros_hbm.at[pl.ds(s * STRIPE, STRIPE)],
                    deg_sh.at[pl.ds(s * STRIPE, STRIPE)])
    pltpu.sync_copy(ones_hbm, ones_v)
    pltpu.sync_copy(dst_hbm.at[c, s], idx_v)
    plsc.subcore_barrier()

    def body(j, carry):
        pltpu.sync_copy(ones_v, deg_sh.at[idx_v.at[j]], add=True)
        return carry

    lax.fori_loop(0, 40, body, 0)
    plsc.subcore_barrier()
    pltpu.sync_copy(deg_sh.at[pl.ds(s * STRIPE, STRIPE)],
                    out_hbm.at[c, pl.ds(s * STRIPE, STRIPE)])


@functools.lru_cache(maxsize=None)
def _deg_call():
    return pl.kernel(
        _deg_body,
        mesh=_mesh(),
        out_type=jax.ShapeDtypeStruct((2, NPAD), jnp.float32),
        scratch_types=[
            pltpu.VMEM((40, CH), jnp.int32),
            pltpu.VMEM((CH,), jnp.float32),
            pltpu.VMEM_SHARED((NPAD,), jnp.float32),
            pltpu.SemaphoreType.DMA,
        ],
    )


def _pipelined_segsum(table_hbm, srcv, dstv, rows, acc_sh, gsems, ssems,
                      nchunks):
    del ssems
    nb = 2

    def gather(j, b):
        return pltpu.async_copy(table_hbm.at[srcv.at[j]], rows.at[b],
                                gsems[b])

    gather(0, 0)

    def body(jj, carry):
        for b in range(nb):
            j = jj * nb + b

            @pl.when(j + 1 < nchunks)
            def _():
                gather(j + 1, 1 - b)

            pltpu.make_async_copy(table_hbm.at[srcv.at[j]], rows.at[b],
                                  gsems[b]).wait()
            pltpu.sync_copy(rows.at[b], acc_sh.at[dstv.at[j]], add=True)
        return carry

    lax.fori_loop(0, nchunks // nb, body, 0)


NROW = 2
NIDX = 8


def _conv1_body(g1_hbm, src_hbm, dst_hbm, out_hbm, ibuf, rows, acc_sh, *sems):
    gsems, isems = sems[:NROW], sems[NROW:]
    c = lax.axis_index("c")
    s = lax.axis_index("s")
    base = c * NPAD + s * STRIPE
    pltpu.sync_copy(g1_hbm.at[pl.ds(base, STRIPE)],
                    acc_sh.at[pl.ds(s * STRIPE, STRIPE)])
    plsc.subcore_barrier()

    nchunks = EP // 16 // CH1

    def load_idx(j, b):
        pltpu.async_copy(src_hbm.at[c, s, j], ibuf.at[b, 0], isems[b])
        pltpu.async_copy(dst_hbm.at[s, j], ibuf.at[b, 1], isems[b])

    def wait_idx(b):
        pltpu.make_async_copy(src_hbm.at[c, s, 0], ibuf.at[b, 0],
                              isems[b]).wait()
        pltpu.make_async_copy(dst_hbm.at[s, 0], ibuf.at[b, 1],
                              isems[b]).wait()

    def gather(br, bi):
        pltpu.async_copy(g1_hbm.at[ibuf.at[bi, 0]], rows.at[br], gsems[br])

    def wait_gather(br, bi):
        pltpu.make_async_copy(g1_hbm.at[ibuf.at[bi, 0]], rows.at[br],
                              gsems[br]).wait()

    for t in range(NIDX):
        load_idx(t, t)
    for t in range(NROW):
        wait_idx(t)
        gather(t, t)

    def body(jj, carry):
        for u in range(NIDX):
            j = jj * NIDX + u
            br, bi, bn = u % NROW, u, (u + NROW) % NIDX
            wait_gather(br, bi)
            pltpu.sync_copy(rows.at[br], acc_sh.at[ibuf.at[bi, 1]], add=True)

            @pl.when(j + NROW < nchunks)
            def _():
                wait_idx(bn)
                gather(br, bn)

            @pl.when(j + NIDX < nchunks)
            def _():
                load_idx(j + NIDX, bi)
        return carry

    lax.fori_loop(0, nchunks // NIDX, body, 0)
    plsc.subcore_barrier()
    pltpu.sync_copy(acc_sh.at[pl.ds(s * STRIPE, STRIPE)],
                    out_hbm.at[pl.ds(base, STRIPE)])


@functools.lru_cache(maxsize=None)
def _conv1_call():
    return pl.kernel(
        _conv1_body,
        mesh=_mesh(),
        out_type=jax.ShapeDtypeStruct((2 * NPAD, HALF), jnp.float32),
        scratch_types=[
            pltpu.VMEM((NIDX, 2, CH1), jnp.int32),
            pltpu.VMEM((NROW, CH1, HALF), jnp.float32),
            pltpu.VMEM_SHARED((NPAD, HALF), jnp.float32),
        ] + [pltpu.SemaphoreType.DMA] * (NROW + NIDX),
    )


def _conv2_body(g2_hbm, z16_hbm, src_hbm, dst_hbm, out_hbm, srcv, dstv, rows,
                acc_sh, gsem0, gsem1, ssem0, ssem1):
    c = lax.axis_index("c")
    s = lax.axis_index("s")

    @pl.when(c == 0)
    def _():
        pltpu.sync_copy(g2_hbm.at[pl.ds(s * STRIPE, STRIPE)],
                        acc_sh.at[pl.ds(s * STRIPE, STRIPE)])

    @pl.when(c == 1)
    def _():
        pltpu.sync_copy(z16_hbm.at[pl.ds(s * STRIPE, STRIPE)],
                        acc_sh.at[pl.ds(s * STRIPE, STRIPE)])

    pltpu.sync_copy(src_hbm.at[c, s], srcv)
    pltpu.sync_copy(dst_hbm.at[c, s], dstv)
    plsc.subcore_barrier()
    _pipelined_segsum(g2_hbm, srcv, dstv, rows, acc_sh, (gsem0, gsem1),
                      (ssem0, ssem1), EP // 32 // CH)
    plsc.subcore_barrier()
    pltpu.sync_copy(acc_sh.at[pl.ds(s * STRIPE, STRIPE)],
                    out_hbm.at[c, pl.ds(s * STRIPE, STRIPE)])


@functools.lru_cache(maxsize=None)
def _conv2_call():
    return pl.kernel(
        _conv2_body,
        mesh=_mesh(),
        out_type=jax.ShapeDtypeStruct((2, NPAD, DOUT), jnp.float32),
        scratch_types=[
            pltpu.VMEM((EP // 32 // CH, CH), jnp.int32),
            pltpu.VMEM((EP // 32 // CH, CH), jnp.int32),
            pltpu.VMEM((2, CH, DOUT), jnp.float32),
            pltpu.VMEM_SHARED((NPAD, DOUT), jnp.float32),
            pltpu.SemaphoreType.DMA,
            pltpu.SemaphoreType.DMA,
            pltpu.SemaphoreType.DMA,
            pltpu.SemaphoreType.DMA,
        ],
        compiler_params=pltpu.CompilerParams(use_tc_tiling_on_sc=False),
    )


RB = 512


def _dinv(degp_ref):
    deg = degp_ref[0] + degp_ref[1] + 1.0
    return lax.rsqrt(deg)[:, None]


def _stageB_body(x_ref, degp_ref, Win_ref, bin_ref, W1_ref, out_ref):
    dinv = _dinv(degp_ref)
    h0 = jnp.maximum(x_ref[...] @ Win_ref[...] + bin_ref[...], 0.0)
    t = (h0 @ W1_ref[...]) * dinv
    out_ref[0] = t[:, :HALF]
    out_ref[1] = t[:, HALF:]


def _stageB(x_pad, degp, W_in, b_in, W1):
    return pl.pallas_call(
        _stageB_body,
        grid=(NPAD // RB,),
        in_specs=[
            pl.BlockSpec((RB, D), lambda i: (i, 0)),
            pl.BlockSpec((2, RB), lambda i: (0, i)),
            pl.BlockSpec((D, D), lambda i: (0, 0)),
            pl.BlockSpec((1, D), lambda i: (0, 0)),
            pl.BlockSpec((D, D), lambda i: (0, 0)),
        ],
        out_specs=pl.BlockSpec((2, RB, HALF), lambda i: (0, i, 0)),
        out_shape=jax.ShapeDtypeStruct((2, NPAD, HALF), jnp.float32),
    )(x_pad, degp, W_in, b_in.reshape(1, D), W1)


def _stageD_body(agg_ref, degp_ref, b1_ref, W2_ref, out_ref):
    dinv = _dinv(degp_ref)
    agg = jnp.concatenate([agg_ref[0], agg_ref[1]], axis=1)
    h1 = jnp.maximum(agg * dinv + b1_ref[...], 0.0)
    out_ref[...] = (h1 @ W2_ref[...]) * dinv


def _stageD(agg1, degp, b1, W2):
    return pl.pallas_call(
        _stageD_body,
        grid=(NPAD // RB,),
        in_specs=[
            pl.BlockSpec((2, RB, HALF), lambda i: (0, i, 0)),
            pl.BlockSpec((2, RB), lambda i: (0, i)),
            pl.BlockSpec((1, D), lambda i: (0, 0)),
            pl.BlockSpec((D, DOUT), lambda i: (0, 0)),
        ],
        out_specs=pl.BlockSpec((RB, DOUT), lambda i: (i, 0)),
        out_shape=jax.ShapeDtypeStruct((NPAD, DOUT), jnp.float32),
    )(agg1, degp, b1.reshape(1, D), W2)


def _stageF_body(acc_ref, degp_ref, b2_ref, out_ref):
    dinv = _dinv(degp_ref)
    out_ref[...] = (acc_ref[0] + acc_ref[1]) * dinv + b2_ref[...]


def _stageF(acc, degp, b2):
    return pl.pallas_call(
        _stageF_body,
        grid=(NPAD // RB,),
        in_specs=[
            pl.BlockSpec((2, RB, DOUT), lambda i: (0, i, 0)),
            pl.BlockSpec((2, RB), lambda i: (0, i)),
            pl.BlockSpec((1, DOUT), lambda i: (0, 0)),
        ],
        out_specs=pl.BlockSpec((RB, DOUT), lambda i: (i, 0)),
        out_shape=jax.ShapeDtypeStruct((NPAD, DOUT), jnp.float32),
    )(acc, degp, b2.reshape(1, DOUT))


def kernel(x, edge_index, W_in, b_in, W1, b1, W2, b2):
    ei = edge_index.astype(jnp.int32)
    src, dst = ei[0], ei[1]
    srcp = jnp.concatenate([src, jnp.full((EP - E,), N, jnp.int32)])
    dstp = jnp.concatenate([dst, jnp.full((EP - E,), N + 200, jnp.int32)])

    x_pad = jnp.pad(x, ((0, NPAD - N), (0, 0)))

    dstA = dstp.reshape(2, 16, 40, CH)
    srcC = jnp.stack([srcp, srcp + NPAD]).reshape(2, 16, EP // 16 // CH1, CH1)
    dstC = dstp.reshape(16, EP // 16 // CH1, CH1)
    srcE = srcp.reshape(2, 16, 40, CH)

    zeros1 = jnp.zeros((NPAD,), jnp.float32)
    ones128 = jnp.ones((CH,), jnp.float32)
    zeros16 = jnp.zeros((NPAD, DOUT), jnp.float32)

    degp = _deg_call()(dstA, zeros1, ones128)
    g1 = _stageB(x_pad, degp, W_in, b_in, W1)
    agg1 = _conv1_call()(g1.reshape(2 * NPAD, HALF), srcC, dstC)
    g2 = _stageD(agg1.reshape(2, NPAD, HALF), degp, b1, W2)
    acc = _conv2_call()(g2, zeros16, srcE, dstA)
    out = _stageF(acc, degp, b2)
    return out[:N]


# speedup vs baseline: 12.4146x; 1.0813x over previous
import functools

import jax
import jax.numpy as jnp
from jax import lax
from jax.experimental import pallas as pl
from jax.experimental.pallas import tpu as pltpu
from jax.experimental.pallas import tpu_sc as plsc

N = 10000
NPAD = 10240
D = 256
HALF = 128
DOUT = 16
E = 160000
EP = 163840
STRIPE = NPAD // 16
CH = 128
CH1 = 64

def _mesh():
    return plsc.VectorSubcoreMesh(core_axis_name="c", subcore_axis_name="s")


def _deg_body(dst_hbm, zeros_hbm, ones_hbm, out_hbm, idx_v, ones_v, deg_sh, sem):
    c = lax.axis_index("c")
    s = lax.axis_index("s")
    pltpu.sync_copy(zeros_hbm.at[pl.ds(s * STRIPE, STRIPE)],
                    deg_sh.at[pl.ds(s * STRIPE, STRIPE)])
    pltpu.sync_copy(ones_hbm, ones_v)
    pltpu.sync_copy(dst_hbm.at[c, s], idx_v)
    plsc.subcore_barrier()

    def body(j, carry):
        pltpu.async_copy(ones_v, deg_sh.at[idx_v.at[j]], sem, add=True)
        return carry

    lax.fori_loop(0, 40, body, 0)

    def drain(j, carry):
        pltpu.make_async_copy(ones_v, deg_sh.at[idx_v.at[0]], sem).wait()
        return carry

    lax.fori_loop(0, 40, drain, 0)
    plsc.subcore_barrier()
    pltpu.sync_copy(deg_sh.at[pl.ds(s * STRIPE, STRIPE)],
                    out_hbm.at[c, pl.ds(s * STRIPE, STRIPE)])


@functools.lru_cache(maxsize=None)
def _deg_call():
    return pl.kernel(
        _deg_body,
        mesh=_mesh(),
        out_type=jax.ShapeDtypeStruct((2, NPAD), jnp.float32),
        scratch_types=[
            pltpu.VMEM((40, CH), jnp.int32),
            pltpu.VMEM((CH,), jnp.float32),
            pltpu.VMEM_SHARED((NPAD,), jnp.float32),
            pltpu.SemaphoreType.DMA,
        ],
    )


def _pipelined_segsum(table_hbm, srcv, dstv, rows, acc_sh, gsems, nchunks):
    nb = len(gsems)

    def gather(j, b):
        return pltpu.async_copy(table_hbm.at[srcv.at[j]], rows.at[b],
                                gsems[b])

    for t in range(nb):
        gather(t, t)

    def body(jj, carry):
        for b in range(nb):
            j = jj * nb + b
            pltpu.make_async_copy(table_hbm.at[srcv.at[j]], rows.at[b],
                                  gsems[b]).wait()
            pltpu.sync_copy(rows.at[b], acc_sh.at[dstv.at[j]], add=True)

            @pl.when(j + nb < nchunks)
            def _():
                gather(j + nb, b)
        return carry

    lax.fori_loop(0, nchunks // nb, body, 0)


NROW = 4
NIDX = 8


def _conv1_body(g1_hbm, src_hbm, dst_hbm, out_hbm, ibuf, rows, acc_sh, *sems):
    gsems, isems = sems[:NROW], sems[NROW:]
    c = lax.axis_index("c")
    s = lax.axis_index("s")
    base = c * NPAD + s * STRIPE
    pltpu.sync_copy(g1_hbm.at[pl.ds(base, STRIPE)],
                    acc_sh.at[pl.ds(s * STRIPE, STRIPE)])
    plsc.subcore_barrier()

    nchunks = EP // 16 // CH1

    def load_idx(j, b):
        pltpu.async_copy(src_hbm.at[c, s, j], ibuf.at[b, 0], isems[b])
        pltpu.async_copy(dst_hbm.at[s, j], ibuf.at[b, 1], isems[b])

    def wait_idx(b):
        pltpu.make_async_copy(src_hbm.at[c, s, 0], ibuf.at[b, 0],
                              isems[b]).wait()
        pltpu.make_async_copy(dst_hbm.at[s, 0], ibuf.at[b, 1],
                              isems[b]).wait()

    def gather(br, bi):
        pltpu.async_copy(g1_hbm.at[ibuf.at[bi, 0]], rows.at[br], gsems[br])

    def wait_gather(br, bi):
        pltpu.make_async_copy(g1_hbm.at[ibuf.at[bi, 0]], rows.at[br],
                              gsems[br]).wait()

    for t in range(NIDX):
        load_idx(t, t)
    for t in range(NROW):
        wait_idx(t)
        gather(t, t)

    def body(jj, carry):
        for u in range(NIDX):
            j = jj * NIDX + u
            br, bi, bn = u % NROW, u, (u + NROW) % NIDX
            wait_gather(br, bi)
            pltpu.sync_copy(rows.at[br], acc_sh.at[ibuf.at[bi, 1]], add=True)

            @pl.when(j + NROW < nchunks)
            def _():
                wait_idx(bn)
                gather(br, bn)

            @pl.when(j + NIDX < nchunks)
            def _():
                load_idx(j + NIDX, bi)
        return carry

    lax.fori_loop(0, nchunks // NIDX, body, 0)
    plsc.subcore_barrier()
    pltpu.sync_copy(acc_sh.at[pl.ds(s * STRIPE, STRIPE)],
                    out_hbm.at[pl.ds(base, STRIPE)])


@functools.lru_cache(maxsize=None)
def _conv1_call():
    return pl.kernel(
        _conv1_body,
        mesh=_mesh(),
        out_type=jax.ShapeDtypeStruct((2 * NPAD, HALF), jnp.float32),
        scratch_types=[
            pltpu.VMEM((NIDX, 2, CH1), jnp.int32),
            pltpu.VMEM((NROW, CH1, HALF), jnp.float32),
            pltpu.VMEM_SHARED((NPAD, HALF), jnp.float32),
        ] + [pltpu.SemaphoreType.DMA] * (NROW + NIDX),
    )


def _conv2_body(g2_hbm, z16_hbm, src_hbm, dst_hbm, out_hbm, srcv, dstv, rows,
                acc_sh, *gsems):
    c = lax.axis_index("c")
    s = lax.axis_index("s")

    @pl.when(c == 0)
    def _():
        pltpu.sync_copy(g2_hbm.at[pl.ds(s * STRIPE, STRIPE)],
                        acc_sh.at[pl.ds(s * STRIPE, STRIPE)])

    @pl.when(c == 1)
    def _():
        pltpu.sync_copy(z16_hbm.at[pl.ds(s * STRIPE, STRIPE)],
                        acc_sh.at[pl.ds(s * STRIPE, STRIPE)])

    pltpu.sync_copy(src_hbm.at[c, s], srcv)
    pltpu.sync_copy(dst_hbm.at[c, s], dstv)
    plsc.subcore_barrier()
    _pipelined_segsum(g2_hbm, srcv, dstv, rows, acc_sh, gsems, EP // 32 // CH)
    plsc.subcore_barrier()
    pltpu.sync_copy(acc_sh.at[pl.ds(s * STRIPE, STRIPE)],
                    out_hbm.at[c, pl.ds(s * STRIPE, STRIPE)])


@functools.lru_cache(maxsize=None)
def _conv2_call():
    return pl.kernel(
        _conv2_body,
        mesh=_mesh(),
        out_type=jax.ShapeDtypeStruct((2, NPAD, DOUT), jnp.float32),
        scratch_types=[
            pltpu.VMEM((EP // 32 // CH, CH), jnp.int32),
            pltpu.VMEM((EP // 32 // CH, CH), jnp.int32),
            pltpu.VMEM((4, CH, DOUT), jnp.float32),
            pltpu.VMEM_SHARED((NPAD, DOUT), jnp.float32),
        ] + [pltpu.SemaphoreType.DMA] * 4,
        compiler_params=pltpu.CompilerParams(use_tc_tiling_on_sc=False),
    )


RB = 512


def _dinv(degp_ref):
    deg = degp_ref[0] + degp_ref[1] + 1.0
    return lax.rsqrt(deg)[:, None]


def _stageB_body(x_ref, degp_ref, Win_ref, bin_ref, W1_ref, out_ref):
    dinv = _dinv(degp_ref)
    h0 = jnp.maximum(x_ref[...] @ Win_ref[...] + bin_ref[...], 0.0)
    t = (h0 @ W1_ref[...]) * dinv
    out_ref[0] = t[:, :HALF]
    out_ref[1] = t[:, HALF:]


def _stageB(x_pad, degp, W_in, b_in, W1):
    return pl.pallas_call(
        _stageB_body,
        grid=(NPAD // RB,),
        in_specs=[
            pl.BlockSpec((RB, D), lambda i: (i, 0)),
            pl.BlockSpec((2, RB), lambda i: (0, i)),
            pl.BlockSpec((D, D), lambda i: (0, 0)),
            pl.BlockSpec((1, D), lambda i: (0, 0)),
            pl.BlockSpec((D, D), lambda i: (0, 0)),
        ],
        out_specs=pl.BlockSpec((2, RB, HALF), lambda i: (0, i, 0)),
        out_shape=jax.ShapeDtypeStruct((2, NPAD, HALF), jnp.float32),
    )(x_pad, degp, W_in, b_in.reshape(1, D), W1)


def _stageD_body(agg_ref, degp_ref, b1_ref, W2_ref, out_ref):
    dinv = _dinv(degp_ref)
    agg = jnp.concatenate([agg_ref[0], agg_ref[1]], axis=1)
    h1 = jnp.maximum(agg * dinv + b1_ref[...], 0.0)
    out_ref[...] = (h1 @ W2_ref[...]) * dinv


def _stageD(agg1, degp, b1, W2):
    return pl.pallas_call(
        _stageD_body,
        grid=(NPAD // RB,),
        in_specs=[
            pl.BlockSpec((2, RB, HALF), lambda i: (0, i, 0)),
            pl.BlockSpec((2, RB), lambda i: (0, i)),
            pl.BlockSpec((1, D), lambda i: (0, 0)),
            pl.BlockSpec((D, DOUT), lambda i: (0, 0)),
        ],
        out_specs=pl.BlockSpec((RB, DOUT), lambda i: (i, 0)),
        out_shape=jax.ShapeDtypeStruct((NPAD, DOUT), jnp.float32),
    )(agg1, degp, b1.reshape(1, D), W2)


def _stageF_body(acc_ref, degp_ref, b2_ref, out_ref):
    dinv = _dinv(degp_ref)
    out_ref[...] = (acc_ref[0] + acc_ref[1]) * dinv + b2_ref[...]


def _stageF(acc, degp, b2):
    return pl.pallas_call(
        _stageF_body,
        grid=(NPAD // RB,),
        in_specs=[
            pl.BlockSpec((2, RB, DOUT), lambda i: (0, i, 0)),
            pl.BlockSpec((2, RB), lambda i: (0, i)),
            pl.BlockSpec((1, DOUT), lambda i: (0, 0)),
        ],
        out_specs=pl.BlockSpec((RB, DOUT), lambda i: (i, 0)),
        out_shape=jax.ShapeDtypeStruct((NPAD, DOUT), jnp.float32),
    )(acc, degp, b2.reshape(1, DOUT))


def kernel(x, edge_index, W_in, b_in, W1, b1, W2, b2):
    ei = edge_index.astype(jnp.int32)
    src, dst = ei[0], ei[1]
    srcp = jnp.concatenate([src, jnp.full((EP - E,), N, jnp.int32)])
    dstp = jnp.concatenate([dst, jnp.full((EP - E,), N + 200, jnp.int32)])

    x_pad = jnp.pad(x, ((0, NPAD - N), (0, 0)))

    dstA = dstp.reshape(2, 16, 40, CH)
    srcC = jnp.stack([srcp, srcp + NPAD]).reshape(2, 16, EP // 16 // CH1, CH1)
    dstC = dstp.reshape(16, EP // 16 // CH1, CH1)
    srcE = srcp.reshape(2, 16, 40, CH)

    zeros1 = jnp.zeros((NPAD,), jnp.float32)
    ones128 = jnp.ones((CH,), jnp.float32)
    zeros16 = jnp.zeros((NPAD, DOUT), jnp.float32)

    degp = _deg_call()(dstA, zeros1, ones128)
    g1 = _stageB(x_pad, degp, W_in, b_in, W1)
    agg1 = _conv1_call()(g1.reshape(2 * NPAD, HALF), srcC, dstC)
    g2 = _stageD(agg1.reshape(2, NPAD, HALF), degp, b1, W2)
    acc = _conv2_call()(g2, zeros16, srcE, dstA)
    out = _stageF(acc, degp, b2)
    return out[:N]
